# baseline (speedup 1.0000x reference)
import functools

import jax
import jax.numpy as jnp
from jax import lax
from jax.experimental import pallas as pl
from jax.experimental.pallas import tpu as pltpu
from jax.experimental.pallas import tpu_sc as plsc

HIDDEN = 2048
TOOL_HID = 512
MAX_TOOLS = 128
TOOL_EMB = 256
TOP_K = 3

TOKEN_BLOCK = 256

SC_CORES = 2
SC_SUBCORES = 16
SC_WORKERS = SC_CORES * SC_SUBCORES
GATHER_CHUNK = 128


def _tc_body(x_ref, wgt_ref, bg_ref, w1_ref, b1_ref, w2_ref, b2_ref,
             probs_ref, gate_ref, idx_ref):
    x = x_ref[...]

    xb = x.astype(jnp.bfloat16).astype(jnp.float32)
    wb = wgt_ref[...].astype(jnp.bfloat16).astype(jnp.float32)
    z = jnp.sum(xb * wb, axis=-1, keepdims=True) + bg_ref[...]
    gate_ref[...] = z > 0.0

    h = jnp.maximum(
        jnp.dot(x, w1_ref[...], preferred_element_type=jnp.float32)
        + b1_ref[...], 0.0)
    logits = (jnp.dot(h, w2_ref[...], preferred_element_type=jnp.float32)
              + b2_ref[...])
    m = jnp.max(logits, axis=-1, keepdims=True)
    e = jnp.exp(logits - m)
    probs = e / jnp.sum(e, axis=-1, keepdims=True)
    probs_ref[...] = probs

    iota = lax.broadcasted_iota(jnp.int32, probs.shape, 1)
    p = probs
    cols = []
    for _ in range(TOP_K):
        pm = jnp.max(p, axis=-1, keepdims=True)
        a = jnp.min(jnp.where(p == pm, iota, MAX_TOOLS), axis=-1,
                    keepdims=True)
        cols.append(a)
        p = jnp.where(iota == a, -1.0, p)
    idx_ref[...] = jnp.concatenate(cols, axis=1)


def _run_tc(x2d, wg_t, b_gate, W_sel1, b_sel1, W_sel2, b_sel2):
    n = x2d.shape[0]
    grid = (n // TOKEN_BLOCK,)
    tok = lambda i: (i, 0)
    rep = lambda i: (0, 0)
    return pl.pallas_call(
        _tc_body,
        grid=grid,
        in_specs=[
            pl.BlockSpec((TOKEN_BLOCK, HIDDEN), tok),
            pl.BlockSpec((1, HIDDEN), rep),
            pl.BlockSpec((1, 1), rep),
            pl.BlockSpec((HIDDEN, TOOL_HID), rep),
            pl.BlockSpec((1, TOOL_HID), rep),
            pl.BlockSpec((TOOL_HID, MAX_TOOLS), rep),
            pl.BlockSpec((1, MAX_TOOLS), rep),
        ],
        out_specs=[
            pl.BlockSpec((TOKEN_BLOCK, MAX_TOOLS), tok),
            pl.BlockSpec((TOKEN_BLOCK, 1), tok),
            pl.BlockSpec((TOKEN_BLOCK, TOP_K), tok),
        ],
        out_shape=[
            jax.ShapeDtypeStruct((n, MAX_TOOLS), jnp.float32),
            jax.ShapeDtypeStruct((n, 1), jnp.bool_),
            jax.ShapeDtypeStruct((n, TOP_K), jnp.int32),
        ],
        compiler_params=pltpu.CompilerParams(
            dimension_semantics=("arbitrary",)),
    )(x2d, wg_t, b_gate, W_sel1, b_sel1, W_sel2, b_sel2)


def _sc_gather(tool_table, idx_flat):
    total = idx_flat.shape[0]
    per_worker = total // SC_WORKERS
    chunks = per_worker // GATHER_CHUNK
    mesh = plsc.VectorSubcoreMesh(core_axis_name="c", subcore_axis_name="s")

    @functools.partial(
        pl.kernel,
        out_type=jax.ShapeDtypeStruct((total, TOOL_EMB), jnp.float32),
        mesh=mesh,
        scratch_types=[
            pltpu.VMEM((per_worker,), jnp.int32),
            pltpu.VMEM((GATHER_CHUNK, TOOL_EMB), jnp.float32),
            pltpu.VMEM((GATHER_CHUNK, TOOL_EMB), jnp.float32),
            pltpu.SemaphoreType.DMA,
            pltpu.SemaphoreType.DMA,
            pltpu.SemaphoreType.DMA,
            pltpu.SemaphoreType.DMA,
        ],
    )
    def gather_kernel(table_hbm, idx_hbm, out_hbm, idx_all, rows0, rows1,
                      sg0, sg1, ss0, ss1):
        wid = lax.axis_index("s") * SC_CORES + lax.axis_index("c")
        base = wid * per_worker
        pltpu.sync_copy(idx_hbm.at[pl.ds(base, per_worker)], idx_all)
        rows = (rows0, rows1)
        sg = (sg0, sg1)
        ss = (ss0, ss1)
        gh, sh = {}, {}
        gh[0] = pltpu.async_copy(
            table_hbm.at[idx_all.at[pl.ds(0, GATHER_CHUNK)]], rows0, sg0)
        for i in range(chunks):
            bi = i & 1
            gh[i].wait()
            sh[i] = pltpu.async_copy(
                rows[bi], out_hbm.at[pl.ds(base + i * GATHER_CHUNK,
                                           GATHER_CHUNK)], ss[bi])
            ni = i + 1
            if ni < chunks:
                nb = ni & 1
                if ni >= 2:
                    sh[ni - 2].wait()
                gh[ni] = pltpu.async_copy(
                    table_hbm.at[idx_all.at[pl.ds(ni * GATHER_CHUNK,
                                                  GATHER_CHUNK)]],
                    rows[nb], sg[nb])
        sh[chunks - 2].wait()
        sh[chunks - 1].wait()

    return gather_kernel(tool_table, idx_flat)


def kernel(hidden_states, W_gate, b_gate, W_sel1, b_sel1, W_sel2, b_sel2,
           tool_table):
    b, s, hdim = hidden_states.shape
    n = b * s
    x2d = hidden_states.reshape(n, hdim)
    wg_t = W_gate.reshape(1, hdim)
    probs, gate, idx = _run_tc(
        x2d, wg_t, b_gate.reshape(1, 1), W_sel1, b_sel1.reshape(1, TOOL_HID),
        W_sel2, b_sel2.reshape(1, MAX_TOOLS))
    embs = _sc_gather(tool_table, idx.reshape(n * TOP_K))
    enh = hidden_states * jnp.float32(1.0)
    return (
        enh,
        probs.reshape(b, s, MAX_TOOLS),
        gate.reshape(b, s, 1),
        idx.reshape(b, s, TOP_K),
        embs.reshape(b, s, TOP_K, TOOL_EMB),
    )


# speedup vs baseline: 1.7478x; 1.7478x over previous
import functools

import jax
import jax.numpy as jnp
from jax import lax
from jax.experimental import pallas as pl
from jax.experimental.pallas import tpu as pltpu
from jax.experimental.pallas import tpu_sc as plsc

HIDDEN = 2048
TOOL_HID = 512
MAX_TOOLS = 128
TOOL_EMB = 256
TOP_K = 3

TOKEN_BLOCK = 256

SC_CORES = 2
SC_SUBCORES = 16
SC_WORKERS = SC_CORES * SC_SUBCORES
GATHER_CHUNK = 128
TABLE_REP = 256


def _tc_body(x_ref, wgt_ref, bg_ref, w1_ref, b1_ref, w2_ref, b2_ref,
             enh_ref, probs_ref, gate_ref, idx_ref, idxs_ref):
    x = x_ref[...]
    enh_ref[...] = x

    xb = x.astype(jnp.bfloat16).astype(jnp.float32)
    wb = wgt_ref[...].astype(jnp.bfloat16).astype(jnp.float32)
    z = jnp.sum(xb * wb, axis=-1, keepdims=True) + bg_ref[...]
    gate_ref[...] = z > 0.0

    h = jnp.maximum(
        jnp.dot(x, w1_ref[...], preferred_element_type=jnp.float32)
        + b1_ref[...], 0.0)
    logits = (jnp.dot(h, w2_ref[...], preferred_element_type=jnp.float32)
              + b2_ref[...])
    m = jnp.max(logits, axis=-1, keepdims=True)
    e = jnp.exp(logits - m)
    probs = e / jnp.sum(e, axis=-1, keepdims=True)
    probs_ref[...] = probs

    iota = lax.broadcasted_iota(jnp.int32, probs.shape, 1)
    p = probs
    cols = []
    for _ in range(TOP_K):
        pm = jnp.max(p, axis=-1, keepdims=True)
        a = jnp.min(jnp.where(p == pm, iota, MAX_TOOLS), axis=-1,
                    keepdims=True)
        cols.append(a)
        p = jnp.where(iota == a, -1.0, p)
    idx = jnp.concatenate(cols, axis=1)
    idx_ref[...] = idx
    r3 = 3 * lax.broadcasted_iota(jnp.int32, idx.shape, 0)
    kk = lax.broadcasted_iota(jnp.int32, idx.shape, 1)
    idxs_ref[...] = idx + MAX_TOOLS * ((r3 + kk) % TABLE_REP)


def _run_tc(x2d, wg_t, b_gate, W_sel1, b_sel1, W_sel2, b_sel2):
    n = x2d.shape[0]
    grid = (n // TOKEN_BLOCK,)
    tok = lambda i: (i, 0)
    rep = lambda i: (0, 0)
    return pl.pallas_call(
        _tc_body,
        grid=grid,
        in_specs=[
            pl.BlockSpec((TOKEN_BLOCK, HIDDEN), tok),
            pl.BlockSpec((1, HIDDEN), rep),
            pl.BlockSpec((1, 1), rep),
            pl.BlockSpec((HIDDEN, TOOL_HID), rep),
            pl.BlockSpec((1, TOOL_HID), rep),
            pl.BlockSpec((TOOL_HID, MAX_TOOLS), rep),
            pl.BlockSpec((1, MAX_TOOLS), rep),
        ],
        out_specs=[
            pl.BlockSpec((TOKEN_BLOCK, HIDDEN), tok),
            pl.BlockSpec((TOKEN_BLOCK, MAX_TOOLS), tok),
            pl.BlockSpec((TOKEN_BLOCK, 1), tok),
            pl.BlockSpec((TOKEN_BLOCK, TOP_K), tok),
            pl.BlockSpec((TOKEN_BLOCK, TOP_K), tok),
        ],
        out_shape=[
            jax.ShapeDtypeStruct((n, HIDDEN), jnp.float32),
            jax.ShapeDtypeStruct((n, MAX_TOOLS), jnp.float32),
            jax.ShapeDtypeStruct((n, 1), jnp.bool_),
            jax.ShapeDtypeStruct((n, TOP_K), jnp.int32),
            jax.ShapeDtypeStruct((n, TOP_K), jnp.int32),
        ],
        compiler_params=pltpu.CompilerParams(
            dimension_semantics=("arbitrary",)),
    )(x2d, wg_t, b_gate, W_sel1, b_sel1, W_sel2, b_sel2)


def _sc_gather(tool_table, idx_flat):
    total = idx_flat.shape[0]
    per_worker = total // SC_WORKERS
    chunks = per_worker // GATHER_CHUNK
    mesh = plsc.VectorSubcoreMesh(core_axis_name="c", subcore_axis_name="s")

    @functools.partial(
        pl.kernel,
        out_type=jax.ShapeDtypeStruct((total, TOOL_EMB), jnp.float32),
        mesh=mesh,
        scratch_types=[
            pltpu.VMEM((per_worker,), jnp.int32),
            pltpu.VMEM((GATHER_CHUNK, TOOL_EMB), jnp.float32),
            pltpu.VMEM((GATHER_CHUNK, TOOL_EMB), jnp.float32),
            pltpu.SemaphoreType.DMA,
            pltpu.SemaphoreType.DMA,
            pltpu.SemaphoreType.DMA,
            pltpu.SemaphoreType.DMA,
        ],
    )
    def gather_kernel(table_hbm, idx_hbm, out_hbm, idx_all, rows0, rows1,
                      sg0, sg1, ss0, ss1):
        wid = lax.axis_index("s") * SC_CORES + lax.axis_index("c")
        base = wid * per_worker
        pltpu.sync_copy(idx_hbm.at[pl.ds(base, per_worker)], idx_all)
        rows = (rows0, rows1)
        sg = (sg0, sg1)
        ss = (ss0, ss1)
        gh, sh = {}, {}
        gh[0] = pltpu.async_copy(
            table_hbm.at[idx_all.at[pl.ds(0, GATHER_CHUNK)]], rows0, sg0)
        for i in range(chunks):
            bi = i & 1
            gh[i].wait()
            sh[i] = pltpu.async_copy(
                rows[bi], out_hbm.at[pl.ds(base + i * GATHER_CHUNK,
                                           GATHER_CHUNK)], ss[bi])
            ni = i + 1
            if ni < chunks:
                nb = ni & 1
                if ni >= 2:
                    sh[ni - 2].wait()
                gh[ni] = pltpu.async_copy(
                    table_hbm.at[idx_all.at[pl.ds(ni * GATHER_CHUNK,
                                                  GATHER_CHUNK)]],
                    rows[nb], sg[nb])
        sh[chunks - 2].wait()
        sh[chunks - 1].wait()

    return gather_kernel(tool_table, idx_flat)


def kernel(hidden_states, W_gate, b_gate, W_sel1, b_sel1, W_sel2, b_sel2,
           tool_table):
    b, s, hdim = hidden_states.shape
    n = b * s
    x2d = hidden_states.reshape(n, hdim)
    wg_t = W_gate.reshape(1, hdim)
    enh, probs, gate, idx, idx_sp = _run_tc(
        x2d, wg_t, b_gate.reshape(1, 1), W_sel1, b_sel1.reshape(1, TOOL_HID),
        W_sel2, b_sel2.reshape(1, MAX_TOOLS))
    table_rep = jnp.tile(tool_table, (TABLE_REP, 1))
    embs = _sc_gather(table_rep, idx_sp.reshape(n * TOP_K))
    return (
        enh.reshape(b, s, hdim),
        probs.reshape(b, s, MAX_TOOLS),
        gate.reshape(b, s, 1),
        idx.reshape(b, s, TOP_K),
        embs.reshape(b, s, TOP_K, TOOL_EMB),
    )
